# baseline (speedup 1.0000x reference)
import functools

import jax
import jax.numpy as jnp
from jax import lax
from jax.experimental import pallas as pl
from jax.experimental.pallas import tpu as pltpu
from jax.experimental.pallas import tpu_sc as plsc

_VOCAB = 100000
_D = 128
_B = 4096
_SEQ = 200
_NCLS = 1000
_NCLS_PAD = 1024

_NC = 2
_NS = 16
_NW = _NC * _NS
_BPW = _B // _NW
_CHUNKS = ((0, 104), (104, 96))
_LANES = 16
_DCH = _D // _LANES


@functools.partial(
    pl.kernel,
    out_type=jax.ShapeDtypeStruct((_B, _D), jnp.float32),
    mesh=plsc.VectorSubcoreMesh(core_axis_name="c", subcore_axis_name="s"),
    scratch_types=[
        pltpu.VMEM((_SEQ,), jnp.int32),
        pltpu.VMEM((_SEQ,), jnp.int32),
        pltpu.VMEM((_SEQ,), jnp.int32),
        pltpu.VMEM((3, _SEQ, _D), jnp.float32),
        pltpu.VMEM((_BPW, _D), jnp.float32),
        pltpu.SemaphoreType.DMA,
        pltpu.SemaphoreType.DMA,
        pltpu.SemaphoreType.DMA,
        pltpu.SemaphoreType.DMA,
        pltpu.SemaphoreType.DMA,
        pltpu.SemaphoreType.DMA,
        pltpu.SemaphoreType.DMA,
        pltpu.SemaphoreType.DMA,
        pltpu.SemaphoreType.DMA,
    ],
)
def _sc_embed_sum(x_hbm, emb_hbm, out_hbm, ir0, ir1, ir2, gbuf, obuf,
                  a0, a1, a2, b0, b1, b2, isem0, isem1, isem2):
    wid = lax.axis_index("s") * _NC + lax.axis_index("c")
    base = wid * _BPW
    csems = ((a0, b0), (a1, b1), (a2, b2))
    isems = (isem0, isem1, isem2)
    idxrows = (ir0, ir1, ir2)
    nbuf = 3

    def idxcopy(i, b):
        return pltpu.make_async_copy(x_hbm.at[base + i], idxrows[b], isems[b])

    def gathers(b):
        return [
            pltpu.make_async_copy(
                emb_hbm.at[idxrows[b].at[pl.ds(off, ln)]],
                gbuf.at[b, pl.ds(off, ln)],
                csems[b][n],
            )
            for n, (off, ln) in enumerate(_CHUNKS)
        ]

    for b in range(nbuf):
        idxcopy(b, b).start()
    for b in range(nbuf):
        idxcopy(b, b).wait()
        for c in gathers(b):
            c.start()

    unroll = 4

    def _reduce(b, t0, nrows, accs):
        def red(j, accs):
            t = t0 + unroll * j
            return tuple(
                a
                + (
                    (
                        gbuf[b, t, pl.ds(k * _LANES, _LANES)]
                        + gbuf[b, t + 1, pl.ds(k * _LANES, _LANES)]
                    )
                    + (
                        gbuf[b, t + 2, pl.ds(k * _LANES, _LANES)]
                        + gbuf[b, t + 3, pl.ds(k * _LANES, _LANES)]
                    )
                )
                for k, a in enumerate(accs)
            )

        return lax.fori_loop(0, nrows // unroll, red, accs)

    def row_body(i, b):
        cs = gathers(b)
        cs[0].wait()

        @pl.when(i + nbuf < _BPW)
        def _():
            idxcopy(i + nbuf, b).start()

        zeros = tuple(jnp.zeros((_LANES,), jnp.float32) for _ in range(_DCH))
        accs = _reduce(b, _CHUNKS[0][0], _CHUNKS[0][1], zeros)
        cs[1].wait()
        accs = _reduce(b, _CHUNKS[1][0], _CHUNKS[1][1], accs)
        for k, a in enumerate(accs):
            obuf[i, pl.ds(k * _LANES, _LANES)] = a

        @pl.when(i + nbuf < _BPW)
        def _():
            idxcopy(i + nbuf, b).wait()
            for c in gathers(b):
                c.start()

    def trip_body(j, carry):
        for b in range(nbuf):
            row_body(nbuf * j + b, b)
        return carry

    lax.fori_loop(0, _BPW // nbuf, trip_body, 0)
    row_body(_BPW - 2, (_BPW - 2) % nbuf)
    row_body(_BPW - 1, (_BPW - 1) % nbuf)
    pltpu.sync_copy(obuf, out_hbm.at[pl.ds(base, _BPW)])


_BM = 512


def _mm_body(e_ref, w_ref, b_ref, o_ref):
    o_ref[...] = (
        lax.dot_general(
            w_ref[...],
            e_ref[...] * (1.0 / _SEQ),
            dimension_numbers=(((1,), (1,)), ((), ())),
            preferred_element_type=jnp.float32,
        )
        + b_ref[...]
    )


def _head_matmul(e_sum, w, bcol):
    return pl.pallas_call(
        _mm_body,
        grid=(_B // _BM,),
        in_specs=[
            pl.BlockSpec((_BM, _D), lambda i: (i, 0)),
            pl.BlockSpec((_NCLS, _D), lambda i: (0, 0)),
            pl.BlockSpec((_NCLS, 1), lambda i: (0, 0)),
        ],
        out_specs=pl.BlockSpec((_NCLS, _BM), lambda i: (0, i)),
        out_shape=jax.ShapeDtypeStruct((_NCLS, _B), jnp.float32),
    )(e_sum, w, bcol)


def kernel(x, emb, W, b):
    e_sum = _sc_embed_sum(x, emb)
    out_t = _head_matmul(e_sum, W, b.reshape(_NCLS, 1))
    return jnp.transpose(out_t)


# speedup vs baseline: 1.0102x; 1.0102x over previous
import functools

import jax
import jax.numpy as jnp
from jax import lax
from jax.experimental import pallas as pl
from jax.experimental.pallas import tpu as pltpu
from jax.experimental.pallas import tpu_sc as plsc

_VOCAB = 100000
_D = 128
_B = 4096
_SEQ = 200
_NCLS = 1000
_NCLS_PAD = 1024

_NC = 2
_NS = 16
_NW = _NC * _NS
_BPW = _B // _NW
_CHUNKS = ((0, 104), (104, 96))
_LANES = 16
_DCH = _D // _LANES


@functools.partial(
    pl.kernel,
    out_type=jax.ShapeDtypeStruct((_B, _D), jnp.float32),
    mesh=plsc.VectorSubcoreMesh(core_axis_name="c", subcore_axis_name="s"),
    scratch_types=[
        pltpu.VMEM((_SEQ,), jnp.int32),
        pltpu.VMEM((_SEQ,), jnp.int32),
        pltpu.VMEM((_SEQ,), jnp.int32),
        pltpu.VMEM((3, _SEQ, _D), jnp.float32),
        pltpu.VMEM((_BPW, _D), jnp.float32),
        pltpu.SemaphoreType.DMA,
        pltpu.SemaphoreType.DMA,
        pltpu.SemaphoreType.DMA,
        pltpu.SemaphoreType.DMA,
        pltpu.SemaphoreType.DMA,
        pltpu.SemaphoreType.DMA,
    ],
)
def _sc_embed_sum(x_hbm, emb_hbm, out_hbm, ir0, ir1, ir2, gbuf, obuf,
                  sem0, sem1, sem2, isem0, isem1, isem2):
    wid = lax.axis_index("s") * _NC + lax.axis_index("c")
    base = wid * _BPW
    sems = (sem0, sem1, sem2)
    isems = (isem0, isem1, isem2)
    idxrows = (ir0, ir1, ir2)
    nbuf = 3

    def idxcopy(i, b):
        return pltpu.make_async_copy(x_hbm.at[base + i], idxrows[b], isems[b])

    def gathers(b):
        return [
            pltpu.make_async_copy(
                emb_hbm.at[idxrows[b].at[pl.ds(off, ln)]],
                gbuf.at[b, pl.ds(off, ln)],
                sems[b],
            )
            for off, ln in _CHUNKS
        ]

    for b in range(nbuf):
        idxcopy(b, b).start()
    for b in range(nbuf):
        idxcopy(b, b).wait()
        for c in gathers(b):
            c.start()

    unroll = 4

    def row_body(i, b):
        for c in gathers(b):
            c.wait()

        @pl.when(i + nbuf < _BPW)
        def _():
            idxcopy(i + nbuf, b).start()

        def red(j, accs):
            t = unroll * j
            return tuple(
                a
                + (
                    (
                        gbuf[b, t, pl.ds(k * _LANES, _LANES)]
                        + gbuf[b, t + 1, pl.ds(k * _LANES, _LANES)]
                    )
                    + (
                        gbuf[b, t + 2, pl.ds(k * _LANES, _LANES)]
                        + gbuf[b, t + 3, pl.ds(k * _LANES, _LANES)]
                    )
                )
                for k, a in enumerate(accs)
            )

        accs = lax.fori_loop(
            0, _SEQ // unroll, red,
            tuple(jnp.zeros((_LANES,), jnp.float32) for _ in range(_DCH)),
        )
        for k, a in enumerate(accs):
            obuf[i, pl.ds(k * _LANES, _LANES)] = a

        @pl.when(i + nbuf < _BPW)
        def _():
            idxcopy(i + nbuf, b).wait()
            for c in gathers(b):
                c.start()

    def trip_body(j, carry):
        for b in range(nbuf):
            row_body(nbuf * j + b, b)
        return carry

    lax.fori_loop(0, _BPW // nbuf, trip_body, 0)
    row_body(_BPW - 2, (_BPW - 2) % nbuf)
    row_body(_BPW - 1, (_BPW - 1) % nbuf)
    pltpu.sync_copy(obuf, out_hbm.at[pl.ds(base, _BPW)])


_BM = 512


def _mm_body(e_ref, w_ref, b_ref, o_ref):
    o_ref[...] = (
        lax.dot_general(
            w_ref[...],
            e_ref[...] * (1.0 / _SEQ),
            dimension_numbers=(((1,), (1,)), ((), ())),
            preferred_element_type=jnp.float32,
        )
        + b_ref[...]
    )


def _head_matmul(e_sum, w, bcol):
    return pl.pallas_call(
        _mm_body,
        grid=(_B // _BM,),
        in_specs=[
            pl.BlockSpec((_BM, _D), lambda i: (i, 0)),
            pl.BlockSpec((_NCLS, _D), lambda i: (0, 0)),
            pl.BlockSpec((_NCLS, 1), lambda i: (0, 0)),
        ],
        out_specs=pl.BlockSpec((_NCLS, _BM), lambda i: (0, i)),
        out_shape=jax.ShapeDtypeStruct((_NCLS, _B), jnp.float32),
    )(e_sum, w, bcol)


def kernel(x, emb, W, b):
    e_sum = _sc_embed_sum(x, emb)
    out_t = _head_matmul(e_sum, W, b.reshape(_NCLS, 1))
    return jnp.transpose(out_t)
